# baseline (speedup 1.0000x reference)
import functools

import jax
import jax.numpy as jnp
from jax import lax
from jax.experimental import pallas as pl
from jax.experimental.pallas import tpu as pltpu
from jax.experimental.pallas import tpu_sc as plsc

B = 16
F = 6
M = 8
K = 10
T = 32768
BT = 4096
NW = F * (K - 1)
NSUB = 16
NCORE = 2
TW = T // (NSUB * NCORE)


def _tc_dense_body(xT_ref, rmat_ref, koff_ref, dmat_ref, bias_ref,
                   lbig_ref, tok_ref):
    x = xT_ref[...]

    xr9 = jnp.dot(rmat_ref[...], x, preferred_element_type=jnp.float32)
    w = jnp.clip(xr9 - koff_ref[...], 0.0, 1.0)
    calib = jnp.dot(dmat_ref[...], w, preferred_element_type=jnp.float32)
    calib = jnp.clip(calib + bias_ref[...], 0.0, 1.0)
    X = [calib[f * M:(f + 1) * M] for f in range(F)]

    a1 = jnp.concatenate([1.0 - X[5], X[5]], 0)
    p2 = jnp.concatenate([a1[:M] * X[4], a1[M:] * X[4]], 0)
    a2 = jnp.concatenate([a1 - p2, p2], 0)
    p3 = jnp.concatenate([a2[i * M:(i + 1) * M] * X[3]
                          for i in range(4)], 0)
    a3 = jnp.concatenate([a2 - p3, p3], 0)

    V = jnp.dot(lbig_ref[...], a3, preferred_element_type=jnp.float32)

    d1 = V[32:] - V[:32]
    e1 = jnp.concatenate([d1[i * M:(i + 1) * M] * X[0]
                          for i in range(4)], 0)
    v32 = V[:32] + e1
    d2 = v32[16:] - v32[:16]
    e2 = jnp.concatenate([d2[:M] * X[1], d2[M:] * X[1]], 0)
    v16 = v32[:16] + e2
    d3 = v16[M:] - v16[:M]
    tok = v16[:M] + d3 * X[2]

    tok_ref[...] = jnp.concatenate(
        [tok, jnp.ones((1, BT), jnp.float32),
         jnp.zeros((16 - M - 1, BT), jnp.float32)], 0)


def _run_tc_dense(xT, rmat, koff, dmat, bias, lbig):
    nblk = T // BT
    return pl.pallas_call(
        _tc_dense_body,
        grid=(nblk,),
        in_specs=[
            pl.BlockSpec((F, BT), lambda i: (0, i)),
            pl.BlockSpec((NW, F), lambda i: (0, 0)),
            pl.BlockSpec((NW, 1), lambda i: (0, 0)),
            pl.BlockSpec((F * M, NW), lambda i: (0, 0)),
            pl.BlockSpec((F * M, 1), lambda i: (0, 0)),
            pl.BlockSpec((64, 64), lambda i: (0, 0)),
        ],
        out_specs=pl.BlockSpec((16, BT), lambda i: (0, i)),
        out_shape=jax.ShapeDtypeStruct((16, T), jnp.float32),
    )(xT, rmat, koff, dmat, bias, lbig)


def _sc_agg_body(tok_hbm, seg_hbm, acc_hbm, rows_v, seg_v, stage_v, acc_sh):
    c = lax.axis_index("c")
    s = lax.axis_index("s")

    @pl.when(s == 0)
    def _():
        for i in range(B):
            stage_v[i] = jnp.zeros((16,), jnp.float32)
        pltpu.sync_copy(stage_v, acc_sh)

    plsc.subcore_barrier()

    base = (c * NSUB + s) * TW
    pltpu.sync_copy(tok_hbm.at[pl.ds(base, TW)], rows_v)
    pltpu.sync_copy(seg_hbm.at[pl.ds(base, TW)], seg_v)
    pltpu.sync_copy(rows_v, acc_sh.at[seg_v], add=True)

    plsc.subcore_barrier()

    @pl.when(s == 0)
    def _():
        pltpu.sync_copy(acc_sh, acc_hbm.at[pl.ds(c * B, B)])


def _make_sc_agg():
    mesh = plsc.VectorSubcoreMesh(core_axis_name="c", subcore_axis_name="s")
    return pl.kernel(
        _sc_agg_body,
        mesh=mesh,
        compiler_params=pltpu.CompilerParams(use_tc_tiling_on_sc=False),
        out_type=jax.ShapeDtypeStruct((NCORE * B, 16), jnp.float32),
        scratch_types=[
            pltpu.VMEM((TW, 16), jnp.float32),
            pltpu.VMEM((TW,), jnp.int32),
            pltpu.VMEM((B, 16), jnp.float32),
            pltpu.VMEM_SHARED((B, 16), jnp.float32),
        ],
    )


def _tc_tail_body(acc_ref, midkT_ref, fin_ref, out_ref):
    acc2 = acc_ref[...]
    acc = acc2[:B] + acc2[B:]
    midkT = midkT_ref[...]
    agg = acc[:, :M] / jnp.maximum(acc[:, M:M + 1], 1.0)
    mid = jnp.zeros((B, M), jnp.float32) + midkT[0:1, :]
    for k in range(K - 1):
        kp = -1.0 + 2.0 * k / (K - 1)
        wmk = jnp.clip((agg - kp) * ((K - 1) / 2.0), 0.0, 1.0)
        mid = mid + wmk * midkT[k + 1:k + 2, :]
    mid = jnp.clip(mid, 0.0, 1.0)
    vals = jnp.zeros((B, 2 ** M), jnp.float32) + fin_ref[...]
    half = (2 ** M) // 2
    for d in range(M):
        vals = (vals[:, :half]
                + (vals[:, half:] - vals[:, :half]) * mid[:, d:d + 1])
        half //= 2
    out_ref[...] = vals


def _run_tc_tail(acc, midkT, finr):
    return pl.pallas_call(
        _tc_tail_body,
        in_specs=[
            pl.BlockSpec((NCORE * B, 16), lambda: (0, 0)),
            pl.BlockSpec((K, M), lambda: (0, 0)),
            pl.BlockSpec((1, 2 ** M), lambda: (0, 0)),
        ],
        out_specs=pl.BlockSpec((B, 1), lambda: (0, 0)),
        out_shape=jax.ShapeDtypeStruct((B, 1), jnp.float32),
    )(acc, midkT, finr)


@jax.jit
def _run(flat, segment_ids, calib_kernel, lattice_kernel, mid_kernel,
         final_kernel):
    xT = flat.T
    seg = segment_ids.astype(jnp.int32)

    frows = jnp.repeat(jnp.arange(F), K - 1)
    krows = jnp.tile(jnp.arange(K - 1), F)
    rmat = 9.0 * jax.nn.one_hot(frows, F, dtype=jnp.float32)
    koff = krows.astype(jnp.float32).reshape(NW, 1)
    deltas = calib_kernel[:, :, 1:]
    dmat = jnp.einsum('mfk,wf,wk->fmw',
                      deltas,
                      jax.nn.one_hot(frows, F, dtype=jnp.float32),
                      jax.nn.one_hot(krows, K - 1, dtype=jnp.float32)
                      ).reshape(F * M, NW)
    bias = calib_kernel[:, :, 0].T.reshape(F * M, 1)
    l3d = lattice_kernel.reshape(M, 8, 8)
    lbig = jnp.einsum('mpq,mn->pmqn', l3d,
                      jnp.eye(M, dtype=jnp.float32)).reshape(64, 64)
    tok16 = _run_tc_dense(xT, rmat, koff, dmat, bias, lbig)
    tok_pad = tok16.T
    acc = _make_sc_agg()(tok_pad, seg)
    return _run_tc_tail(acc, mid_kernel.T, final_kernel.reshape(1, 2 ** M))


def kernel(flat, segment_ids, calib_kernel, lattice_kernel, mid_kernel,
           final_kernel):
    return _run(flat, segment_ids, calib_kernel, lattice_kernel, mid_kernel,
                final_kernel)


# speedup vs baseline: 1.2792x; 1.2792x over previous
import functools

import jax
import jax.numpy as jnp
from jax import lax
from jax.experimental import pallas as pl
from jax.experimental.pallas import tpu as pltpu
from jax.experimental.pallas import tpu_sc as plsc

B = 16
F = 6
M = 8
K = 10
T = 32768
BT = 4096
NW = F * (K - 1)
NSUB = 16
NCORE = 2
CH = 512


def _tc_dense_body(xT_ref, rmat_ref, koff_ref, dmat_ref, bias_ref,
                   lbig_ref, tok_ref):
    x = xT_ref[...]

    xr9 = jnp.dot(rmat_ref[...], x, preferred_element_type=jnp.float32)
    w = jnp.clip(xr9 - koff_ref[...], 0.0, 1.0)
    calib = jnp.dot(dmat_ref[...], w, preferred_element_type=jnp.float32)
    calib = jnp.clip(calib + bias_ref[...], 0.0, 1.0)
    X = [calib[f * M:(f + 1) * M] for f in range(F)]

    a1 = jnp.concatenate([1.0 - X[5], X[5]], 0)
    p2 = jnp.concatenate([a1[:M] * X[4], a1[M:] * X[4]], 0)
    a2 = jnp.concatenate([a1 - p2, p2], 0)
    p3 = jnp.concatenate([a2[i * M:(i + 1) * M] * X[3]
                          for i in range(4)], 0)
    a3 = jnp.concatenate([a2 - p3, p3], 0)

    V = jnp.dot(lbig_ref[...], a3, preferred_element_type=jnp.float32)

    d1 = V[32:] - V[:32]
    e1 = jnp.concatenate([d1[i * M:(i + 1) * M] * X[0]
                          for i in range(4)], 0)
    v32 = V[:32] + e1
    d2 = v32[16:] - v32[:16]
    e2 = jnp.concatenate([d2[:M] * X[1], d2[M:] * X[1]], 0)
    v16 = v32[:16] + e2
    d3 = v16[M:] - v16[:M]
    tok_ref[...] = v16[:M] + d3 * X[2]


def _run_tc_dense(xT, rmat, koff, dmat, bias, lbig):
    nblk = T // BT
    return pl.pallas_call(
        _tc_dense_body,
        grid=(nblk,),
        in_specs=[
            pl.BlockSpec((F, BT), lambda i: (0, i)),
            pl.BlockSpec((NW, F), lambda i: (0, 0)),
            pl.BlockSpec((NW, 1), lambda i: (0, 0)),
            pl.BlockSpec((F * M, NW), lambda i: (0, 0)),
            pl.BlockSpec((F * M, 1), lambda i: (0, 0)),
            pl.BlockSpec((64, 64), lambda i: (0, 0)),
        ],
        out_specs=pl.BlockSpec((M, BT), lambda i: (0, i)),
        out_shape=jax.ShapeDtypeStruct((M, T), jnp.float32),
    )(xT, rmat, koff, dmat, bias, lbig)


def _sc_agg_body(tok_hbm, seg_hbm, acc_hbm, seg_v, buf_v, out_v):
    c = lax.axis_index("c")
    s = lax.axis_index("s")

    pltpu.sync_copy(seg_hbm, seg_v)
    iota = lax.iota(jnp.int32, 16)
    ngrp = T // 16

    def grp_last(g):
        return seg_v[pl.ds(g * 16, 16)][15]

    def bsearch(target):
        gf = jnp.int32(0)
        for sh in range(11, -1, -1):
            cand = gf + jnp.int32(1 << sh)
            val = grp_last(jnp.minimum(cand, ngrp) - 1)
            ok = jnp.logical_and(cand <= ngrp, val < target)
            gf = jnp.where(ok, cand, gf)
        w = seg_v[pl.ds(jnp.minimum(gf, ngrp - 1) * 16, 16)]
        off = jnp.int32(0)
        for lane in range(16):
            off = off + jnp.where(w[lane] < target, 1, 0)
        return jnp.where(gf >= ngrp, jnp.int32(T), gf * 16 + off)

    lo = bsearch(s)
    hi = bsearch(s + 1)
    cnt = hi - lo
    start = lo + c * (cnt // 2)
    end = jnp.where(c == 0, lo + cnt // 2, hi)

    g0 = start // CH
    g1 = jnp.where(end > start, (end + CH - 1) // CH, g0)
    nsub = CH // 16

    def chunk_body(g, accs):
        pos = g * CH
        pltpu.sync_copy(tok_hbm.at[:, pl.ds(pos, CH)], buf_v)
        accs = list(accs)
        for j in range(nsub):
            tix = pos + j * 16 + iota
            mask = jnp.logical_and(tix >= start, tix < end)
            for m in range(M):
                v = buf_v[m, pl.ds(j * 16, 16)]
                accs[m] = accs[m] + jnp.where(mask, v, 0.0)
        return tuple(accs)

    accs0 = tuple(jnp.zeros((16,), jnp.float32) for _ in range(M))
    accs = lax.fori_loop(g0, g1, chunk_body, accs0)

    for m in range(M):
        out_v[pl.ds(m * 16, 16)] = accs[m]
    out_v[pl.ds(M * 16, 16)] = (jnp.zeros((16,), jnp.float32)
                                + cnt.astype(jnp.float32))
    pltpu.sync_copy(out_v, acc_hbm.at[c * B + s])


def _make_sc_agg():
    mesh = plsc.VectorSubcoreMesh(core_axis_name="c", subcore_axis_name="s")
    return pl.kernel(
        _sc_agg_body,
        mesh=mesh,
        compiler_params=pltpu.CompilerParams(use_tc_tiling_on_sc=False),
        out_type=jax.ShapeDtypeStruct((NCORE * B, (M + 1) * 16),
                                      jnp.float32),
        scratch_types=[
            pltpu.VMEM((T,), jnp.int32),
            pltpu.VMEM((M, CH), jnp.float32),
            pltpu.VMEM(((M + 1) * 16,), jnp.float32),
        ],
    )


def _tc_tail_body(acc_ref, midkT_ref, fin_ref, out_ref):
    acc2 = acc_ref[...]
    accl = acc2[:B] + acc2[B:]
    cols = [jnp.sum(accl[:, r * 16:(r + 1) * 16], axis=1, keepdims=True)
            for r in range(M + 1)]
    acc = jnp.concatenate(cols, axis=1)
    acc = acc * jnp.concatenate(
        [jnp.ones((1, M), jnp.float32),
         jnp.full((1, 1), 1.0 / 16.0, jnp.float32)], 1)
    midkT = midkT_ref[...]
    agg = acc[:, :M] / jnp.maximum(acc[:, M:M + 1], 1.0)
    mid = jnp.zeros((B, M), jnp.float32) + midkT[0:1, :]
    for k in range(K - 1):
        kp = -1.0 + 2.0 * k / (K - 1)
        wmk = jnp.clip((agg - kp) * ((K - 1) / 2.0), 0.0, 1.0)
        mid = mid + wmk * midkT[k + 1:k + 2, :]
    mid = jnp.clip(mid, 0.0, 1.0)
    vals = jnp.zeros((B, 2 ** M), jnp.float32) + fin_ref[...]
    half = (2 ** M) // 2
    for d in range(M):
        vals = (vals[:, :half]
                + (vals[:, half:] - vals[:, :half]) * mid[:, d:d + 1])
        half //= 2
    out_ref[...] = vals


def _run_tc_tail(acc, midkT, finr):
    return pl.pallas_call(
        _tc_tail_body,
        in_specs=[
            pl.BlockSpec((NCORE * B, (M + 1) * 16), lambda: (0, 0)),
            pl.BlockSpec((K, M), lambda: (0, 0)),
            pl.BlockSpec((1, 2 ** M), lambda: (0, 0)),
        ],
        out_specs=pl.BlockSpec((B, 1), lambda: (0, 0)),
        out_shape=jax.ShapeDtypeStruct((B, 1), jnp.float32),
    )(acc, midkT, finr)


@jax.jit
def _run(flat, segment_ids, calib_kernel, lattice_kernel, mid_kernel,
         final_kernel):
    xT = flat.T
    seg = segment_ids.astype(jnp.int32)

    frows = jnp.repeat(jnp.arange(F), K - 1)
    krows = jnp.tile(jnp.arange(K - 1), F)
    rmat = 9.0 * jax.nn.one_hot(frows, F, dtype=jnp.float32)
    koff = krows.astype(jnp.float32).reshape(NW, 1)
    deltas = calib_kernel[:, :, 1:]
    dmat = jnp.einsum('mfk,wf,wk->fmw',
                      deltas,
                      jax.nn.one_hot(frows, F, dtype=jnp.float32),
                      jax.nn.one_hot(krows, K - 1, dtype=jnp.float32)
                      ).reshape(F * M, NW)
    bias = calib_kernel[:, :, 0].T.reshape(F * M, 1)
    l3d = lattice_kernel.reshape(M, 8, 8)
    lbig = jnp.einsum('mpq,mn->pmqn', l3d,
                      jnp.eye(M, dtype=jnp.float32)).reshape(64, 64)

    tok_mt = _run_tc_dense(xT, rmat, koff, dmat, bias, lbig)
    acc = _make_sc_agg()(tok_mt, seg)
    return _run_tc_tail(acc, mid_kernel.T, final_kernel.reshape(1, 2 ** M))


def kernel(flat, segment_ids, calib_kernel, lattice_kernel, mid_kernel,
           final_kernel):
    return _run(flat, segment_ids, calib_kernel, lattice_kernel, mid_kernel,
                final_kernel)


# speedup vs baseline: 1.2794x; 1.0002x over previous
import functools

import jax
import jax.numpy as jnp
from jax import lax
from jax.experimental import pallas as pl
from jax.experimental.pallas import tpu as pltpu
from jax.experimental.pallas import tpu_sc as plsc

B = 16
F = 6
M = 8
K = 10
T = 32768
BT = 4096
NW = F * (K - 1)
NSUB = 16
NCORE = 2
CH = 512


def _tc_dense_body(xT_ref, rmat_ref, koff_ref, dmat_ref, bias_ref,
                   lbig_ref, tok_ref):
    x = xT_ref[...]

    xr9 = jnp.dot(rmat_ref[...], x, preferred_element_type=jnp.float32)
    w = jnp.clip(xr9 - koff_ref[...], 0.0, 1.0)
    calib = jnp.dot(dmat_ref[...], w, preferred_element_type=jnp.float32)
    calib = jnp.clip(calib + bias_ref[...], 0.0, 1.0)
    X = [calib[f * M:(f + 1) * M] for f in range(F)]

    a1 = jnp.concatenate([1.0 - X[5], X[5]], 0)
    p2 = jnp.concatenate([a1[:M] * X[4], a1[M:] * X[4]], 0)
    a2 = jnp.concatenate([a1 - p2, p2], 0)
    p3 = jnp.concatenate([a2[i * M:(i + 1) * M] * X[3]
                          for i in range(4)], 0)
    a3 = jnp.concatenate([a2 - p3, p3], 0)

    V = jnp.dot(lbig_ref[...], a3, preferred_element_type=jnp.float32)

    d1 = V[32:] - V[:32]
    e1 = jnp.concatenate([d1[i * M:(i + 1) * M] * X[0]
                          for i in range(4)], 0)
    v32 = V[:32] + e1
    d2 = v32[16:] - v32[:16]
    e2 = jnp.concatenate([d2[:M] * X[1], d2[M:] * X[1]], 0)
    v16 = v32[:16] + e2
    d3 = v16[M:] - v16[:M]
    tok_ref[...] = v16[:M] + d3 * X[2]


def _run_tc_dense(xT, rmat, koff, dmat, bias, lbig):
    nblk = T // BT
    return pl.pallas_call(
        _tc_dense_body,
        grid=(nblk,),
        in_specs=[
            pl.BlockSpec((F, BT), lambda i: (0, i)),
            pl.BlockSpec((NW, F), lambda i: (0, 0)),
            pl.BlockSpec((NW, 1), lambda i: (0, 0)),
            pl.BlockSpec((F * M, NW), lambda i: (0, 0)),
            pl.BlockSpec((F * M, 1), lambda i: (0, 0)),
            pl.BlockSpec((64, 64), lambda i: (0, 0)),
        ],
        out_specs=pl.BlockSpec((M, BT), lambda i: (0, i)),
        out_shape=jax.ShapeDtypeStruct((M, T), jnp.float32),
    )(xT, rmat, koff, dmat, bias, lbig)


def _sc_agg_body(tok_hbm, seg_hbm, acc_hbm, seg_v, buf_v, out_v):
    c = lax.axis_index("c")
    s = lax.axis_index("s")

    pltpu.sync_copy(seg_hbm, seg_v)
    iota = lax.iota(jnp.int32, 16)
    ngrp = T // 16

    def grp_last(g):
        return seg_v[pl.ds(g * 16, 16)][15]

    def bsearch(target):
        gf = jnp.int32(0)
        for sh in range(11, -1, -1):
            cand = gf + jnp.int32(1 << sh)
            val = grp_last(jnp.minimum(cand, ngrp) - 1)
            ok = jnp.logical_and(cand <= ngrp, val < target)
            gf = jnp.where(ok, cand, gf)
        w = seg_v[pl.ds(jnp.minimum(gf, ngrp - 1) * 16, 16)]
        off = jnp.int32(0)
        for lane in range(16):
            off = off + jnp.where(w[lane] < target, 1, 0)
        return jnp.where(gf >= ngrp, jnp.int32(T), gf * 16 + off)

    lo = bsearch(s)
    hi = bsearch(s + 1)
    cnt = hi - lo
    start = lo + c * (cnt // 2)
    end = jnp.where(c == 0, lo + cnt // 2, hi)

    g0 = start // CH
    g1 = jnp.where(end > start, (end + CH - 1) // CH, g0)
    nsub = CH // 16

    def chunk_body(g, accs):
        pos = g * CH
        pltpu.sync_copy(tok_hbm.at[:, pl.ds(pos, CH)], buf_v)
        accs = list(accs)
        for j in range(nsub):
            tix = pos + j * 16 + iota
            mask = jnp.logical_and(tix >= start, tix < end)
            for m in range(M):
                v = buf_v[m, pl.ds(j * 16, 16)]
                accs[m] = accs[m] + jnp.where(mask, v, 0.0)
        return tuple(accs)

    accs0 = tuple(jnp.zeros((16,), jnp.float32) for _ in range(M))
    accs = lax.fori_loop(g0, g1, chunk_body, accs0)

    for m in range(M):
        out_v[pl.ds(m * 16, 16)] = accs[m]
    out_v[pl.ds(M * 16, 16)] = (jnp.zeros((16,), jnp.float32)
                                + (end - start).astype(jnp.float32))
    pltpu.sync_copy(out_v, acc_hbm.at[c * B + s])


def _make_sc_agg():
    mesh = plsc.VectorSubcoreMesh(core_axis_name="c", subcore_axis_name="s")
    return pl.kernel(
        _sc_agg_body,
        mesh=mesh,
        compiler_params=pltpu.CompilerParams(use_tc_tiling_on_sc=False),
        out_type=jax.ShapeDtypeStruct((NCORE * B, (M + 1) * 16),
                                      jnp.float32),
        scratch_types=[
            pltpu.VMEM((T,), jnp.int32),
            pltpu.VMEM((M, CH), jnp.float32),
            pltpu.VMEM(((M + 1) * 16,), jnp.float32),
        ],
    )


def _tc_tail_body(acc_ref, midkT_ref, fin_ref, out_ref):
    acc2 = acc_ref[...]
    accl = acc2[:B] + acc2[B:]
    cols = [jnp.sum(accl[:, r * 16:(r + 1) * 16], axis=1, keepdims=True)
            for r in range(M + 1)]
    acc = jnp.concatenate(cols, axis=1)
    acc = acc * jnp.concatenate(
        [jnp.ones((1, M), jnp.float32),
         jnp.full((1, 1), 1.0 / 16.0, jnp.float32)], 1)
    midkT = midkT_ref[...]
    agg = acc[:, :M] / jnp.maximum(acc[:, M:M + 1], 1.0)
    mid = jnp.zeros((B, M), jnp.float32) + midkT[0:1, :]
    for k in range(K - 1):
        kp = -1.0 + 2.0 * k / (K - 1)
        wmk = jnp.clip((agg - kp) * ((K - 1) / 2.0), 0.0, 1.0)
        mid = mid + wmk * midkT[k + 1:k + 2, :]
    mid = jnp.clip(mid, 0.0, 1.0)
    vals = jnp.zeros((B, 2 ** M), jnp.float32) + fin_ref[...]
    half = (2 ** M) // 2
    for d in range(M):
        vals = (vals[:, :half]
                + (vals[:, half:] - vals[:, :half]) * mid[:, d:d + 1])
        half //= 2
    out_ref[...] = vals


def _run_tc_tail(acc, midkT, finr):
    return pl.pallas_call(
        _tc_tail_body,
        in_specs=[
            pl.BlockSpec((NCORE * B, (M + 1) * 16), lambda: (0, 0)),
            pl.BlockSpec((K, M), lambda: (0, 0)),
            pl.BlockSpec((1, 2 ** M), lambda: (0, 0)),
        ],
        out_specs=pl.BlockSpec((B, 1), lambda: (0, 0)),
        out_shape=jax.ShapeDtypeStruct((B, 1), jnp.float32),
    )(acc, midkT, finr)


@jax.jit
def _run(flat, segment_ids, calib_kernel, lattice_kernel, mid_kernel,
         final_kernel):
    xT = flat.T
    seg = segment_ids.astype(jnp.int32)

    frows = jnp.repeat(jnp.arange(F), K - 1)
    krows = jnp.tile(jnp.arange(K - 1), F)
    rmat = 9.0 * jax.nn.one_hot(frows, F, dtype=jnp.float32)
    koff = krows.astype(jnp.float32).reshape(NW, 1)
    deltas = calib_kernel[:, :, 1:]
    dmat = jnp.einsum('mfk,wf,wk->fmw',
                      deltas,
                      jax.nn.one_hot(frows, F, dtype=jnp.float32),
                      jax.nn.one_hot(krows, K - 1, dtype=jnp.float32)
                      ).reshape(F * M, NW)
    bias = calib_kernel[:, :, 0].T.reshape(F * M, 1)
    l3d = lattice_kernel.reshape(M, 8, 8)
    lbig = jnp.einsum('mpq,mn->pmqn', l3d,
                      jnp.eye(M, dtype=jnp.float32)).reshape(64, 64)

    tok_mt = _run_tc_dense(xT, rmat, koff, dmat, bias, lbig)
    acc = _make_sc_agg()(tok_mt, seg)
    return _run_tc_tail(acc, mid_kernel.T, final_kernel.reshape(1, 2 ** M))


def kernel(flat, segment_ids, calib_kernel, lattice_kernel, mid_kernel,
           final_kernel):
    return _run(flat, segment_ids, calib_kernel, lattice_kernel, mid_kernel,
                final_kernel)
